# Initial kernel scaffold; baseline (speedup 1.0000x reference)
import jax
import jax.numpy as jnp
from jax.experimental import pallas as pl


def kernel(x, edge_attr, W_lin0, b_lin0, We1, be1, We2, be2, W_root, b_conv, Wih_gru, Whh_gru, bih_gru, bhh_gru, Wih_s2s, Whh_s2s, bih_s2s, bhh_s2s, Wih_m, Whh_m, bih_m, bhh_m, Wm1, bm1, Wm2, bm2, edge_index, batch, nonring, nrbidx):
    raise NotImplementedError("write your pallas kernel here")


# speedup vs baseline: 2.8960x; 2.8960x over previous
import jax
import jax.numpy as jnp
from jax import lax
from jax.experimental import pallas as pl
from jax.experimental.pallas import tpu as pltpu
from jax.experimental.pallas import tpu_sc as plsc

NN = 2560
NE = 5120
NG = 128
NPG = NN // NG
TP = 8
H = 64
ACT = 6
NC = 2
NS = 16
NW = NC * NS
W128 = 128
EBLK = 512


def _relu(v):
    return jnp.maximum(v, 0.0)


def _dot(a, b):
    return jnp.dot(a, b, preferred_element_type=jnp.float32)


def _sc_gather(table, idx, width):
    B = idx.shape[0]
    bpw = B // NW
    mesh = plsc.VectorSubcoreMesh(core_axis_name="c", subcore_axis_name="s")

    def body(table_hbm, idx_hbm, out_hbm, idx_v, rows_v, sem):
        wid = lax.axis_index("s") * NC + lax.axis_index("c")
        base = wid * bpw
        pltpu.sync_copy(idx_hbm.at[pl.ds(base, bpw)], idx_v)
        pltpu.async_copy(table_hbm.at[idx_v], rows_v, sem).wait()
        pltpu.sync_copy(rows_v, out_hbm.at[pl.ds(base, bpw)])

    f = pl.kernel(
        body,
        out_type=jax.ShapeDtypeStruct((B, width), jnp.float32),
        mesh=mesh,
        scratch_types=[
            pltpu.VMEM((bpw,), jnp.int32),
            pltpu.VMEM((bpw, width), jnp.float32),
            pltpu.SemaphoreType.DMA,
        ],
    )
    return f(table, idx)


def _sc_scatter_add(values, idx, zeros_rows):
    B, width = values.shape
    n_rows = zeros_rows.shape[0]
    bpw = B // NW
    rpw = n_rows // NS
    mesh = plsc.VectorSubcoreMesh(core_axis_name="c", subcore_axis_name="s")

    def body(val_hbm, idx_hbm, zero_hbm, out_hbm, idx_v, rows_v, acc, sem):
        cid = lax.axis_index("c")
        sid = lax.axis_index("s")
        pltpu.sync_copy(zero_hbm.at[pl.ds(sid * rpw, rpw)],
                        acc.at[pl.ds(sid * rpw, rpw)])
        plsc.subcore_barrier()
        base = (sid * NC + cid) * bpw
        pltpu.sync_copy(idx_hbm.at[pl.ds(base, bpw)], idx_v)
        pltpu.sync_copy(val_hbm.at[pl.ds(base, bpw)], rows_v)
        pltpu.sync_copy(rows_v, acc.at[idx_v], add=True)
        plsc.subcore_barrier()
        pltpu.sync_copy(acc.at[pl.ds(sid * rpw, rpw)],
                        out_hbm.at[cid, pl.ds(sid * rpw, rpw)])

    f = pl.kernel(
        body,
        out_type=jax.ShapeDtypeStruct((NC, n_rows, width), jnp.float32),
        mesh=mesh,
        scratch_types=[
            pltpu.VMEM((bpw,), jnp.int32),
            pltpu.VMEM((bpw, width), jnp.float32),
            pltpu.VMEM_SHARED((n_rows, width), jnp.float32),
            pltpu.SemaphoreType.DMA,
        ],
    )
    return f(values, idx, zeros_rows)


def _tc_prep(x, W_lin0, b_lin0, edge_attr, We1, be1):
    def body(x_ref, wl, bl, ea, we1, be1_, out_ref, f_ref):
        s = _relu(_dot(x_ref[...], wl[...]) + bl[...])
        out_ref[...] = jnp.concatenate([s, jnp.zeros((NN, W128 - H), jnp.float32)],
                                       axis=1)
        f_ref[...] = _relu(_dot(ea[...], we1[...]) + be1_[...])

    return pl.pallas_call(
        body,
        out_shape=(jax.ShapeDtypeStruct((NN, W128), jnp.float32),
                   jax.ShapeDtypeStruct((NE, H), jnp.float32)),
    )(x, W_lin0, b_lin0, edge_attr, We1, be1)


def _tc_msg(f, o_src, M, Be2):
    def body(f_ref, o_ref, m_ref, b_ref, msg_ref):
        fb = f_ref[...]
        ob = o_ref[...][:, :H]
        z = (fb[:, :, None] * ob[:, None, :]).reshape(EBLK, H * H)
        msg = _dot(z, m_ref[...]) + _dot(ob, b_ref[...])
        col = lax.broadcasted_iota(jnp.int32, (EBLK, W128 - H), 1)
        pad = jnp.where(col == 0, 1.0, 0.0)
        msg_ref[...] = jnp.concatenate([msg, pad], axis=1)

    return pl.pallas_call(
        body,
        grid=(NE // EBLK,),
        in_specs=[
            pl.BlockSpec((EBLK, H), lambda i: (i, 0)),
            pl.BlockSpec((EBLK, W128), lambda i: (i, 0)),
            pl.BlockSpec((H * H, H), lambda i: (0, 0)),
            pl.BlockSpec((H, H), lambda i: (0, 0)),
        ],
        out_specs=pl.BlockSpec((EBLK, W128), lambda i: (i, 0)),
        out_shape=jax.ShapeDtypeStruct((NE, W128), jnp.float32),
    )(f, o_src, M, Be2)


def _tc_node_update(s, agg2, W_root, b_conv, Wih, bih, Whh, bhh):
    def body(s_ref, agg_ref, wr, bc, wih, bih_, whh, bhh_, out_ref):
        ag = agg_ref[...]
        both = ag[0] + ag[1]
        agg = both[:, :H]
        deg = both[:, H:H + 1]
        inv = 1.0 / jnp.maximum(deg, 1.0)
        s_ = s_ref[...][:, :H]
        m = _relu(_dot(s_, wr[...]) + agg * inv + bc[...])
        gx = _dot(m, wih[...]) + bih_[...]
        gh = _dot(s_, whh[...]) + bhh_[...]
        r = jax.nn.sigmoid(gx[:, :H] + gh[:, :H])
        zg = jax.nn.sigmoid(gx[:, H:2 * H] + gh[:, H:2 * H])
        n = jnp.tanh(gx[:, 2 * H:] + r * gh[:, 2 * H:])
        s_new = (1.0 - zg) * n + zg * s_
        out_ref[...] = jnp.concatenate(
            [s_new, jnp.zeros((NN, W128 - H), jnp.float32)], axis=1)

    return pl.pallas_call(
        body,
        out_shape=jax.ShapeDtypeStruct((NN, W128), jnp.float32),
    )(s, agg2, W_root, b_conv, Wih, bih, Whh, bhh)


def _tc_final(out_nodes, n_feat, Wih_s2s, Whh_s2s, b_s2s, Wih_m, b_m,
              Wm1a, Wm1b, bm1, Wm2, bm2):
    def body(o_ref, nf_ref, wihs, whhs, bs, wim, bm, w1a, w1b, b1, w2, b2,
             logit_ref, hx_ref, cx_ref):
        o3 = o_ref[...][:, :H].reshape(NG, NPG, H)
        q_star = jnp.zeros((NG, 2 * H), jnp.float32)
        hs = jnp.zeros((NG, H), jnp.float32)
        cs = jnp.zeros((NG, H), jnp.float32)
        for _ in range(6):
            gates = _dot(q_star, wihs[...]) + _dot(hs, whhs[...]) + bs[...]
            ig = jax.nn.sigmoid(gates[:, :H])
            fg = jax.nn.sigmoid(gates[:, H:2 * H])
            gg = jnp.tanh(gates[:, 2 * H:3 * H])
            og = jax.nn.sigmoid(gates[:, 3 * H:])
            cs = fg * cs + ig * gg
            hs = og * jnp.tanh(cs)
            e = jnp.sum(o3 * hs[:, None, :], axis=2)
            ex = jnp.exp(e - jnp.max(e, axis=1, keepdims=True))
            a = ex / jnp.sum(ex, axis=1, keepdims=True)
            rg = jnp.sum(a[:, :, None] * o3, axis=1)
            q_star = jnp.concatenate([hs, rg], axis=1)
        gates = _dot(q_star, wim[...]) + bm[...]
        ig = jax.nn.sigmoid(gates[:, :H])
        gg = jnp.tanh(gates[:, 2 * H:3 * H])
        og = jax.nn.sigmoid(gates[:, 3 * H:])
        cx = ig * gg
        hx = og * jnp.tanh(cx)
        hx_ref[...] = hx
        cx_ref[...] = cx
        t = _dot(hx, w1a[...])
        t4 = jnp.broadcast_to(t[:, None, :], (NG, TP, H)).reshape(NG * TP, H)
        hid = _relu(t4 + _dot(nf_ref[...], w1b[...]) + b1[...])
        logit_ref[...] = _dot(hid, w2[...]) + b2[...]

    return pl.pallas_call(
        body,
        out_shape=(jax.ShapeDtypeStruct((NG * TP, ACT), jnp.float32),
                   jax.ShapeDtypeStruct((NG, H), jnp.float32),
                   jax.ShapeDtypeStruct((NG, H), jnp.float32)),
    )(out_nodes, n_feat, Wih_s2s, Whh_s2s, b_s2s, Wih_m, b_m,
      Wm1a, Wm1b, bm1, Wm2, bm2)


def kernel(x, edge_attr, W_lin0, b_lin0, We1, be1, We2, be2, W_root, b_conv,
           Wih_gru, Whh_gru, bih_gru, bhh_gru, Wih_s2s, Whh_s2s, bih_s2s,
           bhh_s2s, Wih_m, Whh_m, bih_m, bhh_m, Wm1, bm1, Wm2, bm2,
           edge_index, batch, nonring, nrbidx):
    f32 = jnp.float32
    src = edge_index[0]
    dst = edge_index[1]
    M = We2.reshape(H * H, H)
    Be2 = be2.reshape(H, H)
    zeros_agg = jnp.zeros((NN, W128), f32)

    s, f = _tc_prep(x, W_lin0, b_lin0.reshape(1, H), edge_attr, We1,
                    be1.reshape(1, H))
    for _ in range(6):
        o_src = _sc_gather(s, src, W128)
        msg = _tc_msg(f, o_src, M, Be2)
        agg2 = _sc_scatter_add(msg, dst, zeros_agg)
        s = _tc_node_update(s, agg2, W_root, b_conv.reshape(1, H),
                            Wih_gru, bih_gru.reshape(1, 3 * H),
                            Whh_gru, bhh_gru.reshape(1, 3 * H))
    nf = _sc_gather(s, nonring.reshape(-1), W128)[:, :H].reshape(NG * TP, 4 * H)
    logits, hx, cx = _tc_final(
        s, nf, Wih_s2s, Whh_s2s, (bih_s2s + bhh_s2s).reshape(1, 4 * H),
        Wih_m, (bih_m + bhh_m).reshape(1, 4 * H),
        Wm1[:H], Wm1[H:], bm1.reshape(1, H), Wm2, bm2.reshape(1, ACT))
    return (logits.reshape(NG, TP, ACT), hx[None], cx[None])


# speedup vs baseline: 2.9488x; 1.0182x over previous
import jax
import jax.numpy as jnp
from jax import lax
from jax.experimental import pallas as pl
from jax.experimental.pallas import tpu as pltpu
from jax.experimental.pallas import tpu_sc as plsc

NN = 2560
NE = 5120
NG = 128
NPG = NN // NG
TP = 8
H = 64
ACT = 6
NC = 2
NS = 16
NW = NC * NS
W128 = 128
EBLK = 512


def _relu(v):
    return jnp.maximum(v, 0.0)


def _dot(a, b):
    return jnp.dot(a, b, preferred_element_type=jnp.float32)


def _sc_gather(table, idx, width):
    B = idx.shape[0]
    bpw = B // NW
    mesh = plsc.VectorSubcoreMesh(core_axis_name="c", subcore_axis_name="s")

    def body(table_hbm, idx_hbm, out_hbm, idx_v, rows_v, sem):
        wid = lax.axis_index("s") * NC + lax.axis_index("c")
        base = wid * bpw
        pltpu.sync_copy(idx_hbm.at[pl.ds(base, bpw)], idx_v)
        pltpu.async_copy(table_hbm.at[idx_v], rows_v, sem).wait()
        pltpu.sync_copy(rows_v, out_hbm.at[pl.ds(base, bpw)])

    f = pl.kernel(
        body,
        out_type=jax.ShapeDtypeStruct((B, width), jnp.float32),
        mesh=mesh,
        scratch_types=[
            pltpu.VMEM((bpw,), jnp.int32),
            pltpu.VMEM((bpw, width), jnp.float32),
            pltpu.SemaphoreType.DMA,
        ],
    )
    return f(table, idx)


def _sc_scatter_add(values, idx, zeros_rows):
    B, width = values.shape
    n_rows = zeros_rows.shape[0]
    bpw = B // NW
    rpw = n_rows // NS
    mesh = plsc.VectorSubcoreMesh(core_axis_name="c", subcore_axis_name="s")

    def body(val_hbm, idx_hbm, zero_hbm, out_hbm, idx_v, rows_v, acc, sem):
        cid = lax.axis_index("c")
        sid = lax.axis_index("s")
        pltpu.sync_copy(zero_hbm.at[pl.ds(sid * rpw, rpw)],
                        acc.at[pl.ds(sid * rpw, rpw)])
        plsc.subcore_barrier()
        base = (sid * NC + cid) * bpw
        pltpu.sync_copy(idx_hbm.at[pl.ds(base, bpw)], idx_v)
        pltpu.sync_copy(val_hbm.at[pl.ds(base, bpw)], rows_v)
        pltpu.sync_copy(rows_v, acc.at[idx_v], add=True)
        plsc.subcore_barrier()
        pltpu.sync_copy(acc.at[pl.ds(sid * rpw, rpw)],
                        out_hbm.at[cid, pl.ds(sid * rpw, rpw)])

    f = pl.kernel(
        body,
        out_type=jax.ShapeDtypeStruct((NC, n_rows, width), jnp.float32),
        mesh=mesh,
        scratch_types=[
            pltpu.VMEM((bpw,), jnp.int32),
            pltpu.VMEM((bpw, width), jnp.float32),
            pltpu.VMEM_SHARED((n_rows, width), jnp.float32),
            pltpu.SemaphoreType.DMA,
        ],
    )
    return f(values, idx, zeros_rows)


def _sc_scatter_gather(values, sidx, gidx, zeros_rows):
    B, width = values.shape
    n_rows = zeros_rows.shape[0]
    bpw = B // NW
    rpw = n_rows // NS
    mesh = plsc.VectorSubcoreMesh(core_axis_name="c", subcore_axis_name="s")

    def body(val_hbm, sidx_hbm, gidx_hbm, zero_hbm, agg_hbm, asrc_hbm,
             idx_v, rows_v, gidx_v, grows_v, acc, sem):
        cid = lax.axis_index("c")
        sid = lax.axis_index("s")
        pltpu.sync_copy(zero_hbm.at[pl.ds(sid * rpw, rpw)],
                        acc.at[pl.ds(sid * rpw, rpw)])
        plsc.subcore_barrier()
        base = (sid * NC + cid) * bpw
        pltpu.sync_copy(sidx_hbm.at[pl.ds(base, bpw)], idx_v)
        pltpu.sync_copy(val_hbm.at[pl.ds(base, bpw)], rows_v)
        pltpu.sync_copy(rows_v, acc.at[idx_v], add=True)
        plsc.subcore_barrier()
        pltpu.sync_copy(acc.at[pl.ds(sid * rpw, rpw)],
                        agg_hbm.at[cid, pl.ds(sid * rpw, rpw)])
        pltpu.sync_copy(gidx_hbm.at[pl.ds(base, bpw)], gidx_v)
        pltpu.async_copy(acc.at[gidx_v], grows_v, sem).wait()
        pltpu.sync_copy(grows_v, asrc_hbm.at[cid, pl.ds(base, bpw)])

    f = pl.kernel(
        body,
        out_type=(jax.ShapeDtypeStruct((NC, n_rows, width), jnp.float32),
                  jax.ShapeDtypeStruct((NC, B, width), jnp.float32)),
        mesh=mesh,
        scratch_types=[
            pltpu.VMEM((bpw,), jnp.int32),
            pltpu.VMEM((bpw, width), jnp.float32),
            pltpu.VMEM((bpw,), jnp.int32),
            pltpu.VMEM((bpw, width), jnp.float32),
            pltpu.VMEM_SHARED((n_rows, width), jnp.float32),
            pltpu.SemaphoreType.DMA,
        ],
    )
    return f(values, sidx, gidx, zeros_rows)


def _tc_prep(x, W_lin0, b_lin0, edge_attr, We1, be1):
    def body(x_ref, wl, bl, ea, we1, be1_, out_ref, f_ref):
        s = _relu(_dot(x_ref[...], wl[...]) + bl[...])
        out_ref[...] = jnp.concatenate([s, jnp.zeros((NN, W128 - H), jnp.float32)],
                                       axis=1)
        f_ref[...] = _relu(_dot(ea[...], we1[...]) + be1_[...])

    return pl.pallas_call(
        body,
        out_shape=(jax.ShapeDtypeStruct((NN, W128), jnp.float32),
                   jax.ShapeDtypeStruct((NE, H), jnp.float32)),
    )(x, W_lin0, b_lin0, edge_attr, We1, be1)


def _tc_msg(f, o_src, M, Be2):
    def body(f_ref, o_ref, m_ref, b_ref, msg_ref):
        fb = f_ref[...]
        ob = o_ref[...][:, :H]
        z = (fb[:, :, None] * ob[:, None, :]).reshape(EBLK, H * H)
        msg = _dot(z.astype(jnp.bfloat16), m_ref[...]) + _dot(ob, b_ref[...])
        col = lax.broadcasted_iota(jnp.int32, (EBLK, W128 - H), 1)
        pad = jnp.where(col == 0, 1.0, 0.0)
        msg_ref[...] = jnp.concatenate([msg, pad], axis=1)

    return pl.pallas_call(
        body,
        grid=(NE // EBLK,),
        in_specs=[
            pl.BlockSpec((EBLK, H), lambda i: (i, 0)),
            pl.BlockSpec((EBLK, W128), lambda i: (i, 0)),
            pl.BlockSpec((H * H, H), lambda i: (0, 0)),
            pl.BlockSpec((H, H), lambda i: (0, 0)),
        ],
        out_specs=pl.BlockSpec((EBLK, W128), lambda i: (i, 0)),
        out_shape=jax.ShapeDtypeStruct((NE, W128), jnp.float32),
    )(f, o_src, M, Be2)


def _tc_update_msg(S, T, agg2, asrc2, f, Mb, Be2, W_root, b_conv,
                   Wih, bih, Whh, bhh):
    def body(s_ref, t_ref, agg_ref, asrc_ref, f_ref, m_ref, b2_ref,
             wr, bc, wih, bih_, whh, bhh_, S_out, T_out, msg_out):
        ag = agg_ref[...]
        agN = ag[0] + ag[1]
        asc = asrc_ref[...]
        agE = asc[0] + asc[1]
        s_all = jnp.concatenate([s_ref[...][:, :H], t_ref[...][:, :H]], axis=0)
        agg_all = jnp.concatenate([agN[:, :H], agE[:, :H]], axis=0)
        deg_all = jnp.concatenate([agN[:, H:H + 1], agE[:, H:H + 1]], axis=0)
        inv = 1.0 / jnp.maximum(deg_all, 1.0)
        m = _relu(_dot(s_all, wr[...]) + agg_all * inv + bc[...])
        gx = _dot(m, wih[...]) + bih_[...]
        gh = _dot(s_all, whh[...]) + bhh_[...]
        r = jax.nn.sigmoid(gx[:, :H] + gh[:, :H])
        zg = jax.nn.sigmoid(gx[:, H:2 * H] + gh[:, H:2 * H])
        n = jnp.tanh(gx[:, 2 * H:] + r * gh[:, 2 * H:])
        s_new = (1.0 - zg) * n + zg * s_all
        S_out[...] = jnp.concatenate(
            [s_new[:NN], jnp.zeros((NN, W128 - H), jnp.float32)], axis=1)
        o = s_new[NN:]
        T_out[...] = jnp.concatenate(
            [o, jnp.zeros((NE, W128 - H), jnp.float32)], axis=1)
        f_all = f_ref[...]
        col = lax.broadcasted_iota(jnp.int32, (EBLK, W128 - H), 1)
        pad = jnp.where(col == 0, 1.0, 0.0)
        for i in range(NE // EBLK):
            ob = o[i * EBLK:(i + 1) * EBLK]
            fb = f_all[i * EBLK:(i + 1) * EBLK]
            z = (fb[:, :, None] * ob[:, None, :]).reshape(EBLK, H * H)
            msg = _dot(z.astype(jnp.bfloat16), m_ref[...]) + _dot(ob, b2_ref[...])
            msg_out[pl.ds(i * EBLK, EBLK), :] = jnp.concatenate([msg, pad],
                                                                axis=1)

    return pl.pallas_call(
        body,
        out_shape=(jax.ShapeDtypeStruct((NN, W128), jnp.float32),
                   jax.ShapeDtypeStruct((NE, W128), jnp.float32),
                   jax.ShapeDtypeStruct((NE, W128), jnp.float32)),
    )(S, T, agg2, asrc2, f, Mb, Be2, W_root, b_conv, Wih, bih, Whh, bhh)


def _tc_node_update(s, agg2, W_root, b_conv, Wih, bih, Whh, bhh):
    def body(s_ref, agg_ref, wr, bc, wih, bih_, whh, bhh_, out_ref):
        ag = agg_ref[...]
        both = ag[0] + ag[1]
        agg = both[:, :H]
        deg = both[:, H:H + 1]
        inv = 1.0 / jnp.maximum(deg, 1.0)
        s_ = s_ref[...][:, :H]
        m = _relu(_dot(s_, wr[...]) + agg * inv + bc[...])
        gx = _dot(m, wih[...]) + bih_[...]
        gh = _dot(s_, whh[...]) + bhh_[...]
        r = jax.nn.sigmoid(gx[:, :H] + gh[:, :H])
        zg = jax.nn.sigmoid(gx[:, H:2 * H] + gh[:, H:2 * H])
        n = jnp.tanh(gx[:, 2 * H:] + r * gh[:, 2 * H:])
        s_new = (1.0 - zg) * n + zg * s_
        out_ref[...] = jnp.concatenate(
            [s_new, jnp.zeros((NN, W128 - H), jnp.float32)], axis=1)

    return pl.pallas_call(
        body,
        out_shape=jax.ShapeDtypeStruct((NN, W128), jnp.float32),
    )(s, agg2, W_root, b_conv, Wih, bih, Whh, bhh)


def _tc_final(out_nodes, n_feat, Wih_s2s, Whh_s2s, b_s2s, Wih_m, b_m,
              Wm1a, Wm1b, bm1, Wm2, bm2):
    def body(o_ref, nf_ref, wihs, whhs, bs, wim, bm, w1a, w1b, b1, w2, b2,
             logit_ref, hx_ref, cx_ref):
        o3 = o_ref[...][:, :H].reshape(NG, NPG, H)
        q_star = jnp.zeros((NG, 2 * H), jnp.float32)
        hs = jnp.zeros((NG, H), jnp.float32)
        cs = jnp.zeros((NG, H), jnp.float32)
        for _ in range(6):
            gates = _dot(q_star, wihs[...]) + _dot(hs, whhs[...]) + bs[...]
            ig = jax.nn.sigmoid(gates[:, :H])
            fg = jax.nn.sigmoid(gates[:, H:2 * H])
            gg = jnp.tanh(gates[:, 2 * H:3 * H])
            og = jax.nn.sigmoid(gates[:, 3 * H:])
            cs = fg * cs + ig * gg
            hs = og * jnp.tanh(cs)
            e = jnp.sum(o3 * hs[:, None, :], axis=2)
            ex = jnp.exp(e - jnp.max(e, axis=1, keepdims=True))
            a = ex / jnp.sum(ex, axis=1, keepdims=True)
            rg = jnp.sum(a[:, :, None] * o3, axis=1)
            q_star = jnp.concatenate([hs, rg], axis=1)
        gates = _dot(q_star, wim[...]) + bm[...]
        ig = jax.nn.sigmoid(gates[:, :H])
        gg = jnp.tanh(gates[:, 2 * H:3 * H])
        og = jax.nn.sigmoid(gates[:, 3 * H:])
        cx = ig * gg
        hx = og * jnp.tanh(cx)
        hx_ref[...] = hx
        cx_ref[...] = cx
        t = _dot(hx, w1a[...])
        t4 = jnp.broadcast_to(t[:, None, :], (NG, TP, H)).reshape(NG * TP, H)
        hid = _relu(t4 + _dot(nf_ref[...], w1b[...]) + b1[...])
        logit_ref[...] = _dot(hid, w2[...]) + b2[...]

    return pl.pallas_call(
        body,
        out_shape=(jax.ShapeDtypeStruct((NG * TP, ACT), jnp.float32),
                   jax.ShapeDtypeStruct((NG, H), jnp.float32),
                   jax.ShapeDtypeStruct((NG, H), jnp.float32)),
    )(out_nodes, n_feat, Wih_s2s, Whh_s2s, b_s2s, Wih_m, b_m,
      Wm1a, Wm1b, bm1, Wm2, bm2)


def kernel(x, edge_attr, W_lin0, b_lin0, We1, be1, We2, be2, W_root, b_conv,
           Wih_gru, Whh_gru, bih_gru, bhh_gru, Wih_s2s, Whh_s2s, bih_s2s,
           bhh_s2s, Wih_m, Whh_m, bih_m, bhh_m, Wm1, bm1, Wm2, bm2,
           edge_index, batch, nonring, nrbidx):
    f32 = jnp.float32
    src = edge_index[0]
    dst = edge_index[1]
    Mb = We2.reshape(H * H, H).astype(jnp.bfloat16)
    Be2 = be2.reshape(H, H)
    zeros_agg = jnp.zeros((NN, W128), f32)

    s, f = _tc_prep(x, W_lin0, b_lin0.reshape(1, H), edge_attr, We1,
                    be1.reshape(1, H))
    T = _sc_gather(s, src, W128)
    msg = _tc_msg(f, T, Mb, Be2)
    for t in range(6):
        if t < 5:
            agg2, asrc2 = _sc_scatter_gather(msg, dst, src, zeros_agg)
            s, T, msg = _tc_update_msg(
                s, T, agg2, asrc2, f, Mb, Be2, W_root, b_conv.reshape(1, H),
                Wih_gru, bih_gru.reshape(1, 3 * H),
                Whh_gru, bhh_gru.reshape(1, 3 * H))
        else:
            agg2 = _sc_scatter_add(msg, dst, zeros_agg)
            s = _tc_node_update(s, agg2, W_root, b_conv.reshape(1, H),
                                Wih_gru, bih_gru.reshape(1, 3 * H),
                                Whh_gru, bhh_gru.reshape(1, 3 * H))
    nf = _sc_gather(s, nonring.reshape(-1), W128)[:, :H].reshape(NG * TP, 4 * H)
    logits, hx, cx = _tc_final(
        s, nf, Wih_s2s, Whh_s2s, (bih_s2s + bhh_s2s).reshape(1, 4 * H),
        Wih_m, (bih_m + bhh_m).reshape(1, 4 * H),
        Wm1[:H], Wm1[H:], bm1.reshape(1, H), Wm2, bm2.reshape(1, ACT))
    return (logits.reshape(NG, TP, ACT), hx[None], cx[None])
